# Initial kernel scaffold; baseline (speedup 1.0000x reference)
import jax
import jax.numpy as jnp
from jax.experimental import pallas as pl


def kernel(x, w_comp, w1, b1, w2, b2):
    raise NotImplementedError("write your pallas kernel here")


# speedup vs baseline: 2.2659x; 2.2659x over previous
import functools

import numpy as np
import jax
import jax.numpy as jnp
from jax.experimental import pallas as pl
from jax.experimental.pallas import tpu as pltpu


def _round_up(a, m):
    return (a + m - 1) // m * m


def _stage0_body(B, w_ref, p_ref, o_ref):
    for b in range(B):
        acc = jnp.dot(w_ref[...], p_ref[b],
                      preferred_element_type=jnp.float32,
                      precision=jax.lax.Precision.HIGHEST)
        o_ref[b] = jnp.maximum(acc, 0.0).astype(o_ref.dtype)


def _stage0(x, w_comp):
    N, Cin, D, H, W = x.shape
    k = w_comp.shape[0]
    KD, KH, KW = w_comp.shape[2:]
    Dp, Hp, Wp = D // KD, H // KH, W // KW
    Mn = Dp * Hp * Wp
    K = Cin * KD * KH * KW

    p = x.reshape(N, Cin, Dp, KD, Hp, KH, Wp, KW)
    p = p.transpose(0, 1, 3, 5, 7, 2, 4, 6).reshape(N, K, Mn)
    w = w_comp.reshape(k, K).astype(jnp.float32)

    B = 4 if N % 4 == 0 else 1
    out = pl.pallas_call(
        functools.partial(_stage0_body, B),
        out_shape=jax.ShapeDtypeStruct((N, k, Mn), jnp.float32),
        grid=(N // B,),
        in_specs=[
            pl.BlockSpec((k, K), lambda i: (0, 0)),
            pl.BlockSpec((B, K, Mn), lambda i: (i, 0, 0)),
        ],
        out_specs=pl.BlockSpec((B, k, Mn), lambda i: (i, 0, 0)),
        compiler_params=pltpu.CompilerParams(
            dimension_semantics=("parallel",)),
    )(w, p)
    return out.reshape(N, k, Dp, Hp, Wp)


def _axis_indicators(S, K, pad):
    los = [(p - pad) // S for p in range(S)]
    his = [(p + K - 1 - pad) // S for p in range(S)]
    T = max(h - l for h, l in zip(his, los)) + 1
    ring_lo = max(0, -min(los))
    ring_hi = max(0, max(his))
    A = np.zeros((S, K, T), np.float32)
    for p in range(S):
        for kk in range(K):
            A[p, kk, (p + kk - pad) // S - los[p]] = 1.0
    base = np.array([l + ring_lo for l in los], np.int32)
    return A, base, ring_lo, ring_hi, T


def _phase_body(n_phase, Cout, Cin, ntap, offsets, Lout, w_ref, b_ref, x_ref,
                o_ref):
    for ph in range(n_phase):
        for co in range(Cout):
            acc = jnp.zeros((1, Lout), jnp.float32) + b_ref[co]
            for ci in range(Cin):
                for j in range(ntap):
                    off = offsets[ph][j]
                    seg = x_ref[0, ci:ci + 1, off:off + Lout]
                    acc = acc + w_ref[ph * Cout + co, ci * ntap + j] * seg
            o_ref[0, ph * Cout + co:ph * Cout + co + 1, :] = (
                jnp.maximum(acc, 0.0))


def _ups_conv_relu(x, w, b, scale=(4, 2, 2), padding=(2, 1, 1)):
    N, Cin, Dl, Hl, Wl = x.shape
    Cout = w.shape[0]
    KD, KH, KW = w.shape[2:]
    sd, sh, sw = scale
    pd, ph_, pw = padding

    Ad, based, rlo_d, rhi_d, Td = _axis_indicators(sd, KD, pd)
    Ah, baseh, rlo_h, rhi_h, Th = _axis_indicators(sh, KH, ph_)
    Aw, basew, rlo_w, rhi_w, Tw = _axis_indicators(sw, KW, pw)
    n_phase = sd * sh * sw
    ntap = Td * Th * Tw

    wp = jnp.einsum('oidhw,pdx,shy,twz->pstoixyz',
                    w.astype(jnp.float32), Ad, Ah, Aw,
                    precision=jax.lax.Precision.HIGHEST)
    wp = wp.reshape(n_phase * Cout, Cin * ntap)

    Dq = Dl + rlo_d + rhi_d
    Hq = Hl + rlo_h + rhi_h
    Wq = Wl + rlo_w + rhi_w
    P1, P2 = Hq * Wq, Wq
    Lp = Dq * Hq * Wq
    Lout = _round_up(Lp, 128)

    offsets = []
    for p in range(sd):
        for s in range(sh):
            for t in range(sw):
                offs = tuple(
                    (int(based[p]) + j0) * P1 + (int(baseh[s]) + j1) * P2
                    + (int(basew[t]) + j2)
                    for j0 in range(Td) for j1 in range(Th)
                    for j2 in range(Tw))
                offsets.append(offs)
    max_off = max(max(o) for o in offsets)
    Ls = _round_up(Lout + max_off, 128)

    xp = jnp.pad(x, ((0, 0), (0, 0), (rlo_d, rhi_d), (rlo_h, rhi_h),
                     (rlo_w, rhi_w)))
    xf = xp.reshape(N, Cin, Lp)
    xf = jnp.pad(xf, ((0, 0), (0, 0), (0, Ls - Lp)))

    out = pl.pallas_call(
        functools.partial(_phase_body, n_phase, Cout, Cin, ntap,
                          tuple(offsets), Lout),
        out_shape=jax.ShapeDtypeStruct((N, n_phase * Cout, Lout), jnp.float32),
        grid=(N,),
        in_specs=[
            pl.BlockSpec(memory_space=pltpu.MemorySpace.SMEM),
            pl.BlockSpec(memory_space=pltpu.MemorySpace.SMEM),
            pl.BlockSpec((1, Cin, Ls), lambda n: (n, 0, 0)),
        ],
        out_specs=pl.BlockSpec((1, n_phase * Cout, Lout), lambda n: (n, 0, 0)),
        compiler_params=pltpu.CompilerParams(
            dimension_semantics=("parallel",)),
    )(wp, b.astype(jnp.float32), xf)

    y = out[:, :, :Lp].reshape(N, sd, sh, sw, Cout, Dq, Hq, Wq)
    y = y[:, :, :, :, :, :Dl, :Hl, :Wl]
    y = y.transpose(0, 4, 5, 1, 6, 2, 7, 3)
    return y.reshape(N, Cout, Dl * sd, Hl * sh, Wl * sw)


def kernel(x, w_comp, w1, b1, w2, b2):
    y0 = _stage0(x, w_comp)
    y1 = _ups_conv_relu(y0, w1, b1)
    y2 = _ups_conv_relu(y1, w2, b2)
    return y2
